# baseline (device time: 19275 ns/iter reference)
import jax
import jax.numpy as jnp
from jax import lax
from jax.experimental import pallas as pl
from jax.experimental.pallas import tpu as pltpu

N_DEV = 4
B, SQ, SKV, DH = 2, 256, 256, 64
H_LOC = 4
HD_LOC = H_LOC * DH
D_MODEL = 512
PW = 2 * DH


def kernel(x, Wq, K_ext, V_ext, Wo):
    def body(x_hbm, wq_hbm, k_hbm, v_hbm, wo_hbm, out_ref,
             x_v, wq_v, k_v, v_v, wo_v, ctx_ref, dma_sems,
             send_sems, recv_sems):
        my = lax.axis_index("i")
        left = lax.rem(my + N_DEV - 1, N_DEV)
        right = lax.rem(my + 1, N_DEV)
        diag = lax.rem(my + 2, N_DEV)

        cp_x = pltpu.make_async_copy(x_hbm, x_v, dma_sems.at[0])
        cp_wq = pltpu.make_async_copy(
            wq_hbm.at[:, pl.ds(my * HD_LOC, HD_LOC)], wq_v, dma_sems.at[1])
        cp_k = pltpu.make_async_copy(k_hbm, k_v, dma_sems.at[2])
        cp_v = pltpu.make_async_copy(v_hbm, v_v, dma_sems.at[3])
        cp_wo = pltpu.make_async_copy(wo_hbm, wo_v, dma_sems.at[4])
        for cp in (cp_x, cp_wq, cp_k, cp_v, cp_wo):
            cp.start()

        barrier_sem = pltpu.get_barrier_semaphore()
        for nbr in (left, right, diag):
            pl.semaphore_signal(
                barrier_sem, inc=1,
                device_id=(nbr,), device_id_type=pl.DeviceIdType.MESH,
            )
        pl.semaphore_wait(barrier_sem, 3)

        qb = lax.broadcasted_iota(jnp.int32, (SQ, SKV), 0) // 64
        kb = lax.broadcasted_iota(jnp.int32, (SQ, SKV), 1) // 64
        mask = (qb == kb) | (kb == 0) | (((qb + kb) % 3) == 0)

        scale = 0.125 * 1.4426950408889634

        cp_x.wait()
        cp_wq.wait()
        wq_loc = (wq_v[:, :] * scale).astype(jnp.bfloat16)
        qms = []
        for b in range(B):
            xb = x_v[b, :, :].astype(jnp.bfloat16)
            qms.append(jnp.dot(xb, wq_loc, preferred_element_type=jnp.float32))

        def acc_piece(slot, origin, p, acc):
            row0 = origin * HD_LOC + p * PW
            wo_p = wo_v[pl.ds(row0, PW), :].astype(jnp.bfloat16)
            out = []
            for b in range(B):
                y = jnp.dot(ctx_ref[slot, b, :, p * PW:(p + 1) * PW], wo_p,
                            preferred_element_type=jnp.float32)
                out.append(y if acc is None else acc[b] + y)
            return out

        cp_k.wait()
        cp_v.wait()

        rdmas = []
        acc = None
        for h in range(H_LOC):
            for b in range(B):
                q = qms[b][:, h * DH:(h + 1) * DH].astype(jnp.bfloat16)
                k = k_v[b, :, h, :].astype(jnp.bfloat16)
                s = lax.dot_general(
                    q, k, (((1,), (1,)), ((), ())),
                    preferred_element_type=jnp.float32,
                )
                w = jnp.exp2(jnp.where(mask, s, -1e30))
                wsum = jnp.sum(w, axis=1, keepdims=True)
                v = v_v[b, :, h, :].astype(jnp.bfloat16)
                c = jnp.dot(w.astype(jnp.bfloat16), v,
                            preferred_element_type=jnp.float32) / wsum
                ctx_ref[0, b, :, h * DH:(h + 1) * DH] = c.astype(jnp.bfloat16)
            if h % 2 == 1:
                p = h // 2
                for slot, target in ((1, right), (2, left), (3, diag)):
                    rdma = pltpu.make_async_remote_copy(
                        src_ref=ctx_ref.at[0, :, :, pl.ds(p * PW, PW)],
                        dst_ref=ctx_ref.at[slot, :, :, pl.ds(p * PW, PW)],
                        send_sem=send_sems.at[slot - 1, p],
                        recv_sem=recv_sems.at[slot - 1, p],
                        device_id=(target,),
                        device_id_type=pl.DeviceIdType.MESH,
                    )
                    rdma.start()
                    rdmas.append(rdma)
                if p == 0:
                    cp_wo.wait()
                acc = acc_piece(0, my, p, acc)

        for p in range(H_LOC // 2):
            for slot, origin in ((1, left), (2, right), (3, diag)):
                rdmas[3 * p + slot - 1].wait_recv()
                acc = acc_piece(slot, origin, p, acc)

        for b in range(B):
            out_ref[b, :, :] = acc[b]

        for rdma in rdmas:
            rdma.wait_send()

    return pl.pallas_call(
        body,
        out_shape=jax.ShapeDtypeStruct((B, SQ, D_MODEL), jnp.float32),
        in_specs=[pl.BlockSpec(memory_space=pl.ANY)] * 5,
        out_specs=pl.BlockSpec(memory_space=pltpu.VMEM),
        scratch_shapes=[
            pltpu.VMEM((B, SQ, D_MODEL), jnp.float32),
            pltpu.VMEM((D_MODEL, HD_LOC), jnp.float32),
            pltpu.VMEM((B, SKV, H_LOC, DH), jnp.float32),
            pltpu.VMEM((B, SKV, H_LOC, DH), jnp.float32),
            pltpu.VMEM((N_DEV * HD_LOC, D_MODEL), jnp.float32),
            pltpu.VMEM((N_DEV, B, SQ, HD_LOC), jnp.bfloat16),
            pltpu.SemaphoreType.DMA((5,)),
            pltpu.SemaphoreType.DMA((N_DEV - 1, H_LOC // 2)),
            pltpu.SemaphoreType.DMA((N_DEV - 1, H_LOC // 2)),
        ],
        compiler_params=pltpu.CompilerParams(collective_id=0),
    )(x, Wq, K_ext, V_ext, Wo)
